# baseline (speedup 1.0000x reference)
import functools

import jax
import jax.numpy as jnp
from jax import lax
from jax.experimental import pallas as pl
from jax.experimental.pallas import tpu as pltpu
from jax.experimental.pallas import tpu_sc as plsc

_COLS = 285
_SUB = 128


def _sc_histogram(x):
    B, C = x.shape
    info = plsc.get_sparse_core_info()
    nw = info.num_cores * info.num_subcores
    per_w = B // nw
    n_sub = per_w // _SUB
    mesh = plsc.VectorSubcoreMesh(core_axis_name="c", subcore_axis_name="s")

    @functools.partial(
        pl.kernel,
        out_type=jax.ShapeDtypeStruct((B, _COLS), jnp.float32),
        mesh=mesh,
        scratch_types=[
            pltpu.VMEM((64, C), jnp.int32),
            pltpu.VMEM((64, C), jnp.int32),
            pltpu.VMEM((_SUB, _COLS), jnp.float32),
            pltpu.VMEM((_SUB, _COLS), jnp.float32),
            pltpu.SemaphoreType.DMA,
            pltpu.SemaphoreType.DMA,
            pltpu.SemaphoreType.DMA,
            pltpu.SemaphoreType.DMA,
        ],
        compiler_params=pltpu.CompilerParams(use_tc_tiling_on_sc=False,
                                             needs_layout_passes=False),
    )
    def k(x_hbm, out_hbm, xs_a, xs_b, cnt_a, cnt_b,
          isem_a, isem_b, osem_a, osem_b):
        wid = lax.axis_index("s") * info.num_cores + lax.axis_index("c")
        lanes = lax.iota(jnp.int32, 16)
        ones = jnp.full((16,), 1.0, jnp.float32)
        zeros16 = jnp.zeros((16,), jnp.float32)
        xbufs, isems = (xs_a, xs_b), (isem_a, isem_b)
        cbufs, osems = (cnt_a, cnt_b), (osem_a, osem_b)
        pend_in = [None, None]
        pend_out = [None, None]
        n_half = per_w // 64

        def x_copy(hidx, buf, sem):
            row0 = pl.multiple_of(wid * per_w + hidx * 64, 64)
            return pltpu.async_copy(x_hbm.at[pl.ds(row0, 64), :], buf, sem)

        pend_in[0] = x_copy(0, xbufs[0], isems[0])

        for sc_i in range(n_sub):
            cnt, osem = cbufs[sc_i % 2], osems[sc_i % 2]
            row0 = pl.multiple_of(wid * per_w + sc_i * _SUB, _SUB)
            if pend_out[sc_i % 2] is not None:
                pend_out[sc_i % 2].wait()

            @plsc.parallel_loop(0, _SUB, unroll=4)
            def _(r, cnt=cnt):
                for cg in range(224 // 16):
                    cnt[r, pl.ds(cg * 16, 16)] = zeros16

            for half in range(2):
                hidx = sc_i * 2 + half
                xs = xbufs[hidx % 2]
                pend_in[hidx % 2].wait()
                if hidx + 1 < n_half:
                    nb = (hidx + 1) % 2
                    pend_in[nb] = x_copy(hidx + 1, xbufs[nb], isems[nb])
                groups = [(g * 16 + lanes, half * 64 + g * 16 + lanes)
                          for g in range(4)]
                for seg in range(4):
                    boff = 56 * seg
                    hi = min(boff + 56, C)

                    @plsc.parallel_loop(boff, hi, unroll=(4 if seg < 3 else 2))
                    def _(j, xs=xs, cnt=cnt, boff=boff,
                          skip=(seg == 3), groups=groups):
                        jv = jnp.full((16,), j, jnp.int32)
                        idxs = [plsc.load_gather(xs, [rows, jv])
                                for rows, _ in groups]
                        for (_, crows), idx in zip(groups, idxs):
                            plsc.addupdate_scatter(cnt, [crows, idx + boff],
                                                   ones)
                        if skip:
                            for (_, crows), idx in zip(groups, idxs):
                                plsc.store_scatter(cnt, [crows, jv + 56],
                                                   idx.astype(jnp.float32))

            pend_out[sc_i % 2] = pltpu.async_copy(
                cnt, out_hbm.at[pl.ds(row0, _SUB), :], osem)
        for p in pend_out:
            if p is not None:
                p.wait()

    return k(x)


def _mlp_body(s_ref, pe_ref, he_ref, w1_ref, b1_ref, w2_ref, b2_ref, o_ref):
    bt = s_ref.shape[0]
    pe = pe_ref[...]
    he = he_ref[...]
    w1 = w1_ref[...]
    f32 = jnp.float32
    tt = (((1,), (1,)), ((), ()))
    nn = (((1,), (0,)), ((), ()))
    A1 = lax.dot_general(pe, w1[:, 0:12], tt, preferred_element_type=f32)
    A2 = lax.dot_general(pe, w1[:, 12:24], tt, preferred_element_type=f32)
    A3 = lax.dot_general(pe, w1[:, 24:36], tt, preferred_element_type=f32)
    A4 = lax.dot_general(he, w1[:, 36:56], tt, preferred_element_type=f32)
    s = s_ref[...]
    hp = (lax.dot_general(s[:, 0:56], A1, nn, preferred_element_type=f32)
          + lax.dot_general(s[:, 56:112], A2, nn, preferred_element_type=f32)
          + lax.dot_general(s[:, 112:168], A3, nn, preferred_element_type=f32)
          + lax.dot_general(s[:, 168:222], A4[0:54], nn,
                            preferred_element_type=f32)
          + b1_ref[...])
    h = jnp.maximum(hp, 0.0)
    out = lax.dot_general(h, w2_ref[...], tt, preferred_element_type=f32)
    skip = jnp.concatenate(
        [s[:, 224:278], jnp.zeros((bt, 1), jnp.float32)], axis=1)
    o_ref[...] = out + skip + b2_ref[...]


def _tc_mlp(S, play_emb, hand_emb, W1, b1, W2, b2):
    B = S.shape[0]
    BT = 512
    return pl.pallas_call(
        _mlp_body,
        grid=(B // BT,),
        in_specs=[
            pl.BlockSpec((BT, _COLS), lambda i: (i, 0)),
            pl.BlockSpec((56, 12), lambda i: (0, 0)),
            pl.BlockSpec((54, 20), lambda i: (0, 0)),
            pl.BlockSpec((128, 56), lambda i: (0, 0)),
            pl.BlockSpec((1, 128), lambda i: (0, 0)),
            pl.BlockSpec((55, 128), lambda i: (0, 0)),
            pl.BlockSpec((1, 55), lambda i: (0, 0)),
        ],
        out_specs=pl.BlockSpec((BT, 55), lambda i: (i, 0)),
        out_shape=jax.ShapeDtypeStruct((B, 55), jnp.float32),
    )(S, play_emb, hand_emb, W1, b1.reshape(1, 128), W2, b2.reshape(1, 55))


def kernel(x, play_emb, hand_emb, W1, b1, W2, b2):
    S = _sc_histogram(x)
    return _tc_mlp(S, play_emb, hand_emb, W1, b1, W2, b2)


# speedup vs baseline: 1.1293x; 1.1293x over previous
import functools

import jax
import jax.numpy as jnp
from jax import lax
from jax.experimental import pallas as pl
from jax.experimental.pallas import tpu as pltpu
from jax.experimental.pallas import tpu_sc as plsc

_COLS = 280
_SUB = 128


def _sc_histogram(x, base, nrows):
    _, C = x.shape
    info = plsc.get_sparse_core_info()
    nw = info.num_cores * info.num_subcores
    per_w = nrows // nw
    n_sub = per_w // _SUB
    mesh = plsc.VectorSubcoreMesh(core_axis_name="c", subcore_axis_name="s")

    @functools.partial(
        pl.kernel,
        out_type=jax.ShapeDtypeStruct((nrows, _COLS), jnp.float32),
        mesh=mesh,
        scratch_types=[
            pltpu.VMEM((64, C), jnp.int32),
            pltpu.VMEM((64, C), jnp.int32),
            pltpu.VMEM((_SUB, _COLS), jnp.float32),
            pltpu.VMEM((_SUB, _COLS), jnp.float32),
            pltpu.SemaphoreType.DMA,
            pltpu.SemaphoreType.DMA,
            pltpu.SemaphoreType.DMA,
            pltpu.SemaphoreType.DMA,
        ],
        compiler_params=pltpu.CompilerParams(use_tc_tiling_on_sc=False,
                                             needs_layout_passes=False),
    )
    def k(x_hbm, out_hbm, xs_a, xs_b, cnt_a, cnt_b,
          isem_a, isem_b, osem_a, osem_b):
        wid = lax.axis_index("s") * info.num_cores + lax.axis_index("c")
        lanes = lax.iota(jnp.int32, 16)
        ones = jnp.full((16,), 1.0, jnp.float32)
        zeros16 = jnp.zeros((16,), jnp.float32)
        xbufs, isems = (xs_a, xs_b), (isem_a, isem_b)
        cbufs, osems = (cnt_a, cnt_b), (osem_a, osem_b)
        pend_in = [None, None]
        pend_out = [None, None]
        n_half = per_w // 64

        def x_copy(hidx, buf, sem):
            row0 = pl.multiple_of(base + wid * per_w + hidx * 64, 64)
            return pltpu.async_copy(x_hbm.at[pl.ds(row0, 64), :], buf, sem)

        pend_in[0] = x_copy(0, xbufs[0], isems[0])

        for sc_i in range(n_sub):
            cnt, osem = cbufs[sc_i % 2], osems[sc_i % 2]
            row0 = pl.multiple_of(wid * per_w + sc_i * _SUB, _SUB)
            if pend_out[sc_i % 2] is not None:
                pend_out[sc_i % 2].wait()

            @plsc.parallel_loop(0, _SUB, unroll=4)
            def _(r, cnt=cnt):
                for cg in range(224 // 16):
                    cnt[r, pl.ds(cg * 16, 16)] = zeros16

            for half in range(2):
                hidx = sc_i * 2 + half
                xs = xbufs[hidx % 2]
                pend_in[hidx % 2].wait()
                if hidx + 1 < n_half:
                    nb = (hidx + 1) % 2
                    pend_in[nb] = x_copy(hidx + 1, xbufs[nb], isems[nb])
                groups = [(g * 16 + lanes, half * 64 + g * 16 + lanes)
                          for g in range(4)]
                for seg in range(4):
                    boff = 56 * seg
                    hi = min(boff + 56, C)

                    @plsc.parallel_loop(boff, hi, unroll=(4 if seg < 3 else 2))
                    def _(j, xs=xs, cnt=cnt, boff=boff,
                          skip=(seg == 3), groups=groups):
                        jv = jnp.full((16,), j, jnp.int32)
                        idxs = [plsc.load_gather(xs, [rows, jv])
                                for rows, _ in groups]
                        for (_, crows), idx in zip(groups, idxs):
                            plsc.addupdate_scatter(cnt, [crows, idx + boff],
                                                   ones)
                        if skip:
                            for (_, crows), idx in zip(groups, idxs):
                                plsc.store_scatter(cnt, [crows, jv + 56],
                                                   idx.astype(jnp.float32))

            pend_out[sc_i % 2] = pltpu.async_copy(
                cnt, out_hbm.at[pl.ds(row0, _SUB), :], osem)
        for p in pend_out:
            if p is not None:
                p.wait()

    return k(x)


def _mlp_body(s_ref, pe_ref, he_ref, w1_ref, b1_ref, w2_ref, b2_ref, o_ref):
    bt = s_ref.shape[0]
    pe = pe_ref[...]
    he = he_ref[...]
    w1 = w1_ref[...]
    f32 = jnp.float32
    tt = (((1,), (1,)), ((), ()))
    nn = (((1,), (0,)), ((), ()))
    A1 = lax.dot_general(pe, w1[:, 0:12], tt, preferred_element_type=f32)
    A2 = lax.dot_general(pe, w1[:, 12:24], tt, preferred_element_type=f32)
    A3 = lax.dot_general(pe, w1[:, 24:36], tt, preferred_element_type=f32)
    A4 = lax.dot_general(he, w1[:, 36:56], tt, preferred_element_type=f32)
    s = s_ref[...]
    hp = (lax.dot_general(s[:, 0:56], A1, nn, preferred_element_type=f32)
          + lax.dot_general(s[:, 56:112], A2, nn, preferred_element_type=f32)
          + lax.dot_general(s[:, 112:168], A3, nn, preferred_element_type=f32)
          + lax.dot_general(s[:, 168:222], A4[0:54], nn,
                            preferred_element_type=f32)
          + b1_ref[...])
    h = jnp.maximum(hp, 0.0)
    out = lax.dot_general(h, w2_ref[...], tt, preferred_element_type=f32)
    skip = jnp.concatenate(
        [s[:, 224:278], jnp.zeros((bt, 1), jnp.float32)], axis=1)
    o_ref[...] = out + skip + b2_ref[...]


def _tc_mlp(S, play_emb, hand_emb, W1, b1, W2, b2):
    B = S.shape[0]
    BT = 512
    return pl.pallas_call(
        _mlp_body,
        grid=(B // BT,),
        in_specs=[
            pl.BlockSpec((BT, _COLS), lambda i: (i, 0)),
            pl.BlockSpec((56, 12), lambda i: (0, 0)),
            pl.BlockSpec((54, 20), lambda i: (0, 0)),
            pl.BlockSpec((128, 56), lambda i: (0, 0)),
            pl.BlockSpec((1, 128), lambda i: (0, 0)),
            pl.BlockSpec((55, 128), lambda i: (0, 0)),
            pl.BlockSpec((1, 55), lambda i: (0, 0)),
        ],
        out_specs=pl.BlockSpec((BT, 55), lambda i: (i, 0)),
        out_shape=jax.ShapeDtypeStruct((B, 55), jnp.float32),
    )(S, play_emb, hand_emb, W1, b1.reshape(1, 128), W2, b2.reshape(1, 55))


def kernel(x, play_emb, hand_emb, W1, b1, W2, b2):
    B = x.shape[0]
    n_chunk = 2
    cb = B // n_chunk
    outs = []
    for i in range(n_chunk):
        S = _sc_histogram(x, i * cb, cb)
        outs.append(_tc_mlp(S, play_emb, hand_emb, W1, b1, W2, b2))
    return jnp.concatenate(outs, axis=0)


# speedup vs baseline: 1.1560x; 1.0236x over previous
import functools

import jax
import jax.numpy as jnp
from jax import lax
from jax.experimental import pallas as pl
from jax.experimental.pallas import tpu as pltpu
from jax.experimental.pallas import tpu_sc as plsc

_COLS = 280
_SUB = 128


def _sc_histogram(x, base, nrows):
    _, C = x.shape
    info = plsc.get_sparse_core_info()
    nw = info.num_cores * info.num_subcores
    per_w = nrows // nw
    n_sub = per_w // _SUB
    mesh = plsc.VectorSubcoreMesh(core_axis_name="c", subcore_axis_name="s")

    @functools.partial(
        pl.kernel,
        out_type=jax.ShapeDtypeStruct((nrows, _COLS), jnp.float32),
        mesh=mesh,
        scratch_types=[
            pltpu.VMEM((64, C), jnp.int32),
            pltpu.VMEM((64, C), jnp.int32),
            pltpu.VMEM((_SUB, _COLS), jnp.float32),
            pltpu.VMEM((_SUB, _COLS), jnp.float32),
            pltpu.SemaphoreType.DMA,
            pltpu.SemaphoreType.DMA,
            pltpu.SemaphoreType.DMA,
            pltpu.SemaphoreType.DMA,
        ],
        compiler_params=pltpu.CompilerParams(use_tc_tiling_on_sc=False,
                                             needs_layout_passes=False),
    )
    def k(x_hbm, out_hbm, xs_a, xs_b, cnt_a, cnt_b,
          isem_a, isem_b, osem_a, osem_b):
        wid = lax.axis_index("s") * info.num_cores + lax.axis_index("c")
        lanes = lax.iota(jnp.int32, 16)
        ones = jnp.full((16,), 1.0, jnp.float32)
        zeros16 = jnp.zeros((16,), jnp.float32)
        xbufs, isems = (xs_a, xs_b), (isem_a, isem_b)
        cbufs, osems = (cnt_a, cnt_b), (osem_a, osem_b)
        pend_in = [None, None]
        pend_out = [None, None]
        n_half = per_w // 64

        def x_copy(hidx, buf, sem):
            row0 = pl.multiple_of(base + wid * per_w + hidx * 64, 64)
            return pltpu.async_copy(x_hbm.at[pl.ds(row0, 64), :], buf, sem)

        pend_in[0] = x_copy(0, xbufs[0], isems[0])

        for sc_i in range(n_sub):
            cnt, osem = cbufs[sc_i % 2], osems[sc_i % 2]
            row0 = pl.multiple_of(wid * per_w + sc_i * _SUB, _SUB)
            if pend_out[sc_i % 2] is not None:
                pend_out[sc_i % 2].wait()

            @plsc.parallel_loop(0, _SUB, unroll=4)
            def _(r, cnt=cnt):
                for cg in range(224 // 16):
                    cnt[r, pl.ds(cg * 16, 16)] = zeros16

            for half in range(2):
                hidx = sc_i * 2 + half
                xs = xbufs[hidx % 2]
                pend_in[hidx % 2].wait()
                if hidx + 1 < n_half:
                    nb = (hidx + 1) % 2
                    pend_in[nb] = x_copy(hidx + 1, xbufs[nb], isems[nb])
                groups = [(g * 16 + lanes, half * 64 + g * 16 + lanes)
                          for g in range(4)]
                for seg in range(4):
                    boff = 56 * seg
                    hi = min(boff + 56, C)

                    @plsc.parallel_loop(boff, hi, unroll=(4 if seg < 3 else 2))
                    def _(j, xs=xs, cnt=cnt, boff=boff,
                          skip=(seg == 3), groups=groups):
                        jv = jnp.full((16,), j, jnp.int32)
                        idxs = [plsc.load_gather(xs, [rows, jv])
                                for rows, _ in groups]
                        for (_, crows), idx in zip(groups, idxs):
                            plsc.addupdate_scatter(cnt, [crows, idx + boff],
                                                   ones)
                        if skip:
                            for (_, crows), idx in zip(groups, idxs):
                                plsc.store_scatter(cnt, [crows, jv + 56],
                                                   idx.astype(jnp.float32))

            pend_out[sc_i % 2] = pltpu.async_copy(
                cnt, out_hbm.at[pl.ds(row0, _SUB), :], osem)
        for p in pend_out:
            if p is not None:
                p.wait()

    return k(x)


def _mlp_body(s_ref, pe_ref, he_ref, w1_ref, b1_ref, w2_ref, b2_ref, o_ref):
    bt = s_ref.shape[0]
    pe = pe_ref[...]
    he = he_ref[...]
    w1 = w1_ref[...]
    f32 = jnp.float32
    tt = (((1,), (1,)), ((), ()))
    nn = (((1,), (0,)), ((), ()))
    A1 = lax.dot_general(pe, w1[:, 0:12], tt, preferred_element_type=f32)
    A2 = lax.dot_general(pe, w1[:, 12:24], tt, preferred_element_type=f32)
    A3 = lax.dot_general(pe, w1[:, 24:36], tt, preferred_element_type=f32)
    A4 = lax.dot_general(he, w1[:, 36:56], tt, preferred_element_type=f32)
    s = s_ref[...]
    hp = (lax.dot_general(s[:, 0:56], A1, nn, preferred_element_type=f32)
          + lax.dot_general(s[:, 56:112], A2, nn, preferred_element_type=f32)
          + lax.dot_general(s[:, 112:168], A3, nn, preferred_element_type=f32)
          + lax.dot_general(s[:, 168:222], A4[0:54], nn,
                            preferred_element_type=f32)
          + b1_ref[...])
    h = jnp.maximum(hp, 0.0)
    out = lax.dot_general(h, w2_ref[...], tt, preferred_element_type=f32)
    skip = jnp.concatenate(
        [s[:, 224:278], jnp.zeros((bt, 1), jnp.float32)], axis=1)
    o_ref[...] = out + skip + b2_ref[...]


def _tc_mlp(S, play_emb, hand_emb, W1, b1, W2, b2):
    B = S.shape[0]
    BT = 512
    return pl.pallas_call(
        _mlp_body,
        grid=(B // BT,),
        in_specs=[
            pl.BlockSpec((BT, _COLS), lambda i: (i, 0)),
            pl.BlockSpec((56, 12), lambda i: (0, 0)),
            pl.BlockSpec((54, 20), lambda i: (0, 0)),
            pl.BlockSpec((128, 56), lambda i: (0, 0)),
            pl.BlockSpec((1, 128), lambda i: (0, 0)),
            pl.BlockSpec((55, 128), lambda i: (0, 0)),
            pl.BlockSpec((1, 55), lambda i: (0, 0)),
        ],
        out_specs=pl.BlockSpec((BT, 55), lambda i: (i, 0)),
        out_shape=jax.ShapeDtypeStruct((B, 55), jnp.float32),
    )(S, play_emb, hand_emb, W1, b1.reshape(1, 128), W2, b2.reshape(1, 55))


def kernel(x, play_emb, hand_emb, W1, b1, W2, b2):
    B = x.shape[0]
    n_chunk = 4
    cb = B // n_chunk
    outs = []
    for i in range(n_chunk):
        S = _sc_histogram(x, i * cb, cb)
        outs.append(_tc_mlp(S, play_emb, hand_emb, W1, b1, W2, b2))
    return jnp.concatenate(outs, axis=0)


# speedup vs baseline: 1.1739x; 1.0155x over previous
import functools

import jax
import jax.numpy as jnp
from jax import lax
from jax.experimental import pallas as pl
from jax.experimental.pallas import tpu as pltpu
from jax.experimental.pallas import tpu_sc as plsc

_COLS = 224
_SUB = 128


def _sc_histogram(x, base, nrows):
    _, C = x.shape
    info = plsc.get_sparse_core_info()
    nw = info.num_cores * info.num_subcores
    per_w = nrows // nw
    n_sub = per_w // _SUB
    mesh = plsc.VectorSubcoreMesh(core_axis_name="c", subcore_axis_name="s")

    @functools.partial(
        pl.kernel,
        out_type=jax.ShapeDtypeStruct((nrows, _COLS), jnp.float32),
        mesh=mesh,
        scratch_types=[
            pltpu.VMEM((64, C), jnp.int32),
            pltpu.VMEM((64, C), jnp.int32),
            pltpu.VMEM((_SUB, _COLS), jnp.float32),
            pltpu.VMEM((_SUB, _COLS), jnp.float32),
            pltpu.SemaphoreType.DMA,
            pltpu.SemaphoreType.DMA,
            pltpu.SemaphoreType.DMA,
            pltpu.SemaphoreType.DMA,
        ],
        compiler_params=pltpu.CompilerParams(use_tc_tiling_on_sc=False,
                                             needs_layout_passes=False),
    )
    def k(x_hbm, out_hbm, xs_a, xs_b, cnt_a, cnt_b,
          isem_a, isem_b, osem_a, osem_b):
        wid = lax.axis_index("s") * info.num_cores + lax.axis_index("c")
        lanes = lax.iota(jnp.int32, 16)
        ones = jnp.full((16,), 1.0, jnp.float32)
        zeros16 = jnp.zeros((16,), jnp.float32)
        xbufs, isems = (xs_a, xs_b), (isem_a, isem_b)
        cbufs, osems = (cnt_a, cnt_b), (osem_a, osem_b)
        pend_in = [None, None]
        pend_out = [None, None]
        n_half = per_w // 64

        def x_copy(hidx, buf, sem):
            row0 = pl.multiple_of(base + wid * per_w + hidx * 64, 64)
            return pltpu.async_copy(x_hbm.at[pl.ds(row0, 64), :], buf, sem)

        pend_in[0] = x_copy(0, xbufs[0], isems[0])

        for sc_i in range(n_sub):
            cnt, osem = cbufs[sc_i % 2], osems[sc_i % 2]
            row0 = pl.multiple_of(wid * per_w + sc_i * _SUB, _SUB)
            if pend_out[sc_i % 2] is not None:
                pend_out[sc_i % 2].wait()

            @plsc.parallel_loop(0, _SUB, unroll=4)
            def _(r, cnt=cnt):
                for cg in range(224 // 16):
                    cnt[r, pl.ds(cg * 16, 16)] = zeros16

            for half in range(2):
                hidx = sc_i * 2 + half
                xs = xbufs[hidx % 2]
                pend_in[hidx % 2].wait()
                if hidx + 1 < n_half:
                    nb = (hidx + 1) % 2
                    pend_in[nb] = x_copy(hidx + 1, xbufs[nb], isems[nb])
                groups = [(g * 16 + lanes, half * 64 + g * 16 + lanes)
                          for g in range(4)]
                for seg in range(4):
                    boff = 56 * seg
                    hi = min(boff + 56, C)

                    @plsc.parallel_loop(boff, hi, unroll=(4 if seg < 3 else 2))
                    def _(j, xs=xs, cnt=cnt, boff=boff, groups=groups):
                        jv = jnp.full((16,), j, jnp.int32)
                        idxs = [plsc.load_gather(xs, [rows, jv])
                                for rows, _ in groups]
                        for (_, crows), idx in zip(groups, idxs):
                            plsc.addupdate_scatter(cnt, [crows, idx + boff],
                                                   ones)

            pend_out[sc_i % 2] = pltpu.async_copy(
                cnt, out_hbm.at[pl.ds(row0, _SUB), :], osem)
        for p in pend_out:
            if p is not None:
                p.wait()

    return k(x)


def _mlp_body(s_ref, x_ref, pe_ref, he_ref, w1_ref, b1_ref, w2_ref, b2_ref,
              o_ref):
    bt = s_ref.shape[0]
    pe = pe_ref[...]
    he = he_ref[...]
    w1 = w1_ref[...]
    f32 = jnp.float32
    tt = (((1,), (1,)), ((), ()))
    nn = (((1,), (0,)), ((), ()))
    A1 = lax.dot_general(pe, w1[:, 0:12], tt, preferred_element_type=f32)
    A2 = lax.dot_general(pe, w1[:, 12:24], tt, preferred_element_type=f32)
    A3 = lax.dot_general(pe, w1[:, 24:36], tt, preferred_element_type=f32)
    A4 = lax.dot_general(he, w1[:, 36:56], tt, preferred_element_type=f32)
    s = s_ref[...]
    hp = (lax.dot_general(s[:, 0:56], A1, nn, preferred_element_type=f32)
          + lax.dot_general(s[:, 56:112], A2, nn, preferred_element_type=f32)
          + lax.dot_general(s[:, 112:168], A3, nn, preferred_element_type=f32)
          + lax.dot_general(s[:, 168:222], A4[0:54], nn,
                            preferred_element_type=f32)
          + b1_ref[...])
    h = jnp.maximum(hp, 0.0)
    out = lax.dot_general(h, w2_ref[...], tt, preferred_element_type=f32)
    a4 = x_ref[:, 168:222].astype(f32)
    skip = jnp.concatenate([a4, jnp.zeros((bt, 1), jnp.float32)], axis=1)
    o_ref[...] = out + skip + b2_ref[...]


def _tc_mlp(S, x, chunk0, play_emb, hand_emb, W1, b1, W2, b2):
    Bc = S.shape[0]
    BT = 512
    co = chunk0 // BT
    return pl.pallas_call(
        _mlp_body,
        grid=(Bc // BT,),
        in_specs=[
            pl.BlockSpec((BT, _COLS), lambda i: (i, 0)),
            pl.BlockSpec((BT, 222), lambda i: (i + co, 0)),
            pl.BlockSpec((56, 12), lambda i: (0, 0)),
            pl.BlockSpec((54, 20), lambda i: (0, 0)),
            pl.BlockSpec((128, 56), lambda i: (0, 0)),
            pl.BlockSpec((1, 128), lambda i: (0, 0)),
            pl.BlockSpec((55, 128), lambda i: (0, 0)),
            pl.BlockSpec((1, 55), lambda i: (0, 0)),
        ],
        out_specs=pl.BlockSpec((BT, 55), lambda i: (i, 0)),
        out_shape=jax.ShapeDtypeStruct((Bc, 55), jnp.float32),
    )(S, x, play_emb, hand_emb, W1, b1.reshape(1, 128), W2,
      b2.reshape(1, 55))


def kernel(x, play_emb, hand_emb, W1, b1, W2, b2):
    B = x.shape[0]
    n_chunk = 4
    cb = B // n_chunk
    outs = []
    for i in range(n_chunk):
        S = _sc_histogram(x, i * cb, cb)
        outs.append(_tc_mlp(S, x, i * cb, play_emb, hand_emb, W1, b1, W2, b2))
    return jnp.concatenate(outs, axis=0)


# speedup vs baseline: 1.4920x; 1.2709x over previous
import functools

import jax
import jax.numpy as jnp
from jax import lax
from jax.experimental import pallas as pl
from jax.experimental.pallas import tpu as pltpu
from jax.experimental.pallas import tpu_sc as plsc

_COLS = 224
_SUB = 128


def _sc_histogram(x, base, nrows):
    _, C = x.shape
    info = plsc.get_sparse_core_info()
    nw = info.num_cores * info.num_subcores
    per_w = nrows // nw
    n_sub = per_w // _SUB
    mesh = plsc.VectorSubcoreMesh(core_axis_name="c", subcore_axis_name="s")

    @functools.partial(
        pl.kernel,
        out_type=jax.ShapeDtypeStruct((nrows, _COLS), jnp.float32),
        mesh=mesh,
        scratch_types=[
            pltpu.VMEM((64, C), jnp.int32),
            pltpu.VMEM((64, C), jnp.int32),
            pltpu.VMEM((_SUB, _COLS), jnp.float32),
            pltpu.VMEM((_SUB, _COLS), jnp.float32),
            pltpu.SemaphoreType.DMA,
            pltpu.SemaphoreType.DMA,
            pltpu.SemaphoreType.DMA,
            pltpu.SemaphoreType.DMA,
        ],
        compiler_params=pltpu.CompilerParams(use_tc_tiling_on_sc=True,
                                             needs_layout_passes=False),
    )
    def k(x_hbm, out_hbm, xs_a, xs_b, cnt_a, cnt_b,
          isem_a, isem_b, osem_a, osem_b):
        wid = lax.axis_index("s") * info.num_cores + lax.axis_index("c")
        lanes = lax.iota(jnp.int32, 16)
        ones = jnp.full((16,), 1.0, jnp.float32)
        zeros16 = jnp.zeros((16,), jnp.float32)
        xbufs, isems = (xs_a, xs_b), (isem_a, isem_b)
        cbufs, osems = (cnt_a, cnt_b), (osem_a, osem_b)
        pend_in = [None, None]
        pend_out = [None, None]
        n_half = per_w // 64

        def x_copy(hidx, buf, sem):
            row0 = pl.multiple_of(base + wid * per_w + hidx * 64, 64)
            return pltpu.async_copy(x_hbm.at[pl.ds(row0, 64), :], buf, sem)

        pend_in[0] = x_copy(0, xbufs[0], isems[0])

        for sc_i in range(n_sub):
            cnt, osem = cbufs[sc_i % 2], osems[sc_i % 2]
            row0 = pl.multiple_of(wid * per_w + sc_i * _SUB, _SUB)
            if pend_out[sc_i % 2] is not None:
                pend_out[sc_i % 2].wait()

            @plsc.parallel_loop(0, _SUB, unroll=4)
            def _(r, cnt=cnt):
                for cg in range(224 // 16):
                    cnt[r, pl.ds(cg * 16, 16)] = zeros16

            for half in range(2):
                hidx = sc_i * 2 + half
                xs = xbufs[hidx % 2]
                pend_in[hidx % 2].wait()
                if hidx + 1 < n_half:
                    nb = (hidx + 1) % 2
                    pend_in[nb] = x_copy(hidx + 1, xbufs[nb], isems[nb])
                groups = [(g * 16 + lanes, half * 64 + g * 16 + lanes)
                          for g in range(4)]
                for seg in range(4):
                    boff = 56 * seg
                    hi = min(boff + 56, C)

                    @plsc.parallel_loop(boff, hi, unroll=(4 if seg < 3 else 2))
                    def _(j, xs=xs, cnt=cnt, boff=boff, groups=groups):
                        jv = jnp.full((16,), j, jnp.int32)
                        idxs = [plsc.load_gather(xs, [rows, jv])
                                for rows, _ in groups]
                        for (_, crows), idx in zip(groups, idxs):
                            plsc.addupdate_scatter(cnt, [crows, idx + boff],
                                                   ones)

            pend_out[sc_i % 2] = pltpu.async_copy(
                cnt, out_hbm.at[pl.ds(row0, _SUB), :], osem)
        for p in pend_out:
            if p is not None:
                p.wait()

    return k(x)


def _mlp_body(s_ref, x_ref, pe_ref, he_ref, w1_ref, b1_ref, w2_ref, b2_ref,
              o_ref):
    bt = s_ref.shape[0]
    pe = pe_ref[...]
    he = he_ref[...]
    w1 = w1_ref[...]
    f32 = jnp.float32
    tt = (((1,), (1,)), ((), ()))
    nn = (((1,), (0,)), ((), ()))
    A1 = lax.dot_general(pe, w1[:, 0:12], tt, preferred_element_type=f32)
    A2 = lax.dot_general(pe, w1[:, 12:24], tt, preferred_element_type=f32)
    A3 = lax.dot_general(pe, w1[:, 24:36], tt, preferred_element_type=f32)
    A4 = lax.dot_general(he, w1[:, 36:56], tt, preferred_element_type=f32)
    s = s_ref[...]
    hp = (lax.dot_general(s[:, 0:56], A1, nn, preferred_element_type=f32)
          + lax.dot_general(s[:, 56:112], A2, nn, preferred_element_type=f32)
          + lax.dot_general(s[:, 112:168], A3, nn, preferred_element_type=f32)
          + lax.dot_general(s[:, 168:222], A4[0:54], nn,
                            preferred_element_type=f32)
          + b1_ref[...])
    h = jnp.maximum(hp, 0.0)
    out = lax.dot_general(h, w2_ref[...], tt, preferred_element_type=f32)
    a4 = x_ref[:, 168:222].astype(f32)
    skip = jnp.concatenate([a4, jnp.zeros((bt, 1), jnp.float32)], axis=1)
    o_ref[...] = out + skip + b2_ref[...]


def _tc_mlp(S, x, chunk0, play_emb, hand_emb, W1, b1, W2, b2):
    Bc = S.shape[0]
    BT = 512
    co = chunk0 // BT
    return pl.pallas_call(
        _mlp_body,
        grid=(Bc // BT,),
        in_specs=[
            pl.BlockSpec((BT, _COLS), lambda i: (i, 0)),
            pl.BlockSpec((BT, 222), lambda i: (i + co, 0)),
            pl.BlockSpec((56, 12), lambda i: (0, 0)),
            pl.BlockSpec((54, 20), lambda i: (0, 0)),
            pl.BlockSpec((128, 56), lambda i: (0, 0)),
            pl.BlockSpec((1, 128), lambda i: (0, 0)),
            pl.BlockSpec((55, 128), lambda i: (0, 0)),
            pl.BlockSpec((1, 55), lambda i: (0, 0)),
        ],
        out_specs=pl.BlockSpec((BT, 55), lambda i: (i, 0)),
        out_shape=jax.ShapeDtypeStruct((Bc, 55), jnp.float32),
    )(S, x, play_emb, hand_emb, W1, b1.reshape(1, 128), W2,
      b2.reshape(1, 55))


def kernel(x, play_emb, hand_emb, W1, b1, W2, b2):
    B = x.shape[0]
    n_chunk = 4
    cb = B // n_chunk
    outs = []
    for i in range(n_chunk):
        S = _sc_histogram(x, i * cb, cb)
        outs.append(_tc_mlp(S, x, i * cb, play_emb, hand_emb, W1, b1, W2, b2))
    return jnp.concatenate(outs, axis=0)


# speedup vs baseline: 1.5530x; 1.0409x over previous
import functools

import jax
import jax.numpy as jnp
from jax import lax
from jax.experimental import pallas as pl
from jax.experimental.pallas import tpu as pltpu
from jax.experimental.pallas import tpu_sc as plsc

_COLS = 224
_SUB = 128


def _sc_histogram(x, base, nrows):
    _, C = x.shape
    info = plsc.get_sparse_core_info()
    nw = info.num_cores * info.num_subcores
    per_w = nrows // nw
    n_sub = per_w // _SUB
    mesh = plsc.VectorSubcoreMesh(core_axis_name="c", subcore_axis_name="s")

    @functools.partial(
        pl.kernel,
        out_type=jax.ShapeDtypeStruct((nrows, _COLS), jnp.float32),
        mesh=mesh,
        scratch_types=[
            pltpu.VMEM((64, C), jnp.int32),
            pltpu.VMEM((64, C), jnp.int32),
            pltpu.VMEM((_SUB, _COLS), jnp.float32),
            pltpu.VMEM((_SUB, _COLS), jnp.float32),
            pltpu.SemaphoreType.DMA,
            pltpu.SemaphoreType.DMA,
            pltpu.SemaphoreType.DMA,
            pltpu.SemaphoreType.DMA,
        ],
        compiler_params=pltpu.CompilerParams(use_tc_tiling_on_sc=True,
                                             needs_layout_passes=False),
    )
    def k(x_hbm, out_hbm, xs_a, xs_b, cnt_a, cnt_b,
          isem_a, isem_b, osem_a, osem_b):
        wid = lax.axis_index("s") * info.num_cores + lax.axis_index("c")
        lanes = lax.iota(jnp.int32, 16)
        ones = jnp.full((16,), 1.0, jnp.float32)
        zeros16 = jnp.zeros((16,), jnp.float32)
        xbufs, isems = (xs_a, xs_b), (isem_a, isem_b)
        cbufs, osems = (cnt_a, cnt_b), (osem_a, osem_b)
        pend_in = [None, None]
        pend_out = [None, None]
        n_half = per_w // 64

        def x_copy(hidx, buf, sem):
            row0 = pl.multiple_of(base + wid * per_w + hidx * 64, 64)
            return pltpu.async_copy(x_hbm.at[pl.ds(row0, 64), :], buf, sem)

        pend_in[0] = x_copy(0, xbufs[0], isems[0])

        for sc_i in range(n_sub):
            cnt, osem = cbufs[sc_i % 2], osems[sc_i % 2]
            row0 = pl.multiple_of(wid * per_w + sc_i * _SUB, _SUB)
            if pend_out[sc_i % 2] is not None:
                pend_out[sc_i % 2].wait()

            @plsc.parallel_loop(0, _SUB, unroll=4)
            def _(r, cnt=cnt):
                for cg in range(224 // 16):
                    cnt[r, pl.ds(cg * 16, 16)] = zeros16

            for half in range(2):
                hidx = sc_i * 2 + half
                xs = xbufs[hidx % 2]
                pend_in[hidx % 2].wait()
                if hidx + 1 < n_half:
                    nb = (hidx + 1) % 2
                    pend_in[nb] = x_copy(hidx + 1, xbufs[nb], isems[nb])
                groups = [(g * 16 + lanes, half * 64 + g * 16 + lanes)
                          for g in range(4)]
                for seg in range(4):
                    boff = 56 * seg
                    hi = min(boff + 56, C)

                    @plsc.parallel_loop(boff, hi, unroll=(4 if seg < 3 else 2))
                    def _(j, xs=xs, cnt=cnt, boff=boff, groups=groups):
                        jv = jnp.full((16,), j, jnp.int32)
                        idxs = [plsc.load_gather(xs, [rows, jv])
                                for rows, _ in groups]
                        for (_, crows), idx in zip(groups, idxs):
                            plsc.addupdate_scatter(cnt, [crows, idx + boff],
                                                   ones)

            pend_out[sc_i % 2] = pltpu.async_copy(
                cnt, out_hbm.at[pl.ds(row0, _SUB), :], osem)
        for p in pend_out:
            if p is not None:
                p.wait()

    return k(x)


def _mlp_body(s_ref, x_ref, pe_ref, he_ref, w1_ref, b1_ref, w2_ref, b2_ref,
              o_ref):
    bt = s_ref.shape[0]
    pe = pe_ref[...]
    he = he_ref[...]
    w1 = w1_ref[...]
    f32 = jnp.float32
    tt = (((1,), (1,)), ((), ()))
    nn = (((1,), (0,)), ((), ()))
    A1 = lax.dot_general(pe, w1[:, 0:12], tt, preferred_element_type=f32)
    A2 = lax.dot_general(pe, w1[:, 12:24], tt, preferred_element_type=f32)
    A3 = lax.dot_general(pe, w1[:, 24:36], tt, preferred_element_type=f32)
    A4 = lax.dot_general(he, w1[:, 36:56], tt, preferred_element_type=f32)
    s = s_ref[...]
    hp = (lax.dot_general(s[:, 0:56], A1, nn, preferred_element_type=f32)
          + lax.dot_general(s[:, 56:112], A2, nn, preferred_element_type=f32)
          + lax.dot_general(s[:, 112:168], A3, nn, preferred_element_type=f32)
          + lax.dot_general(s[:, 168:222], A4[0:54], nn,
                            preferred_element_type=f32)
          + b1_ref[...])
    h = jnp.maximum(hp, 0.0)
    out = lax.dot_general(h, w2_ref[...], tt, preferred_element_type=f32)
    a4 = x_ref[:, 168:222].astype(f32)
    skip = jnp.concatenate([a4, jnp.zeros((bt, 1), jnp.float32)], axis=1)
    o_ref[...] = out + skip + b2_ref[...]


def _tc_mlp(S, x, chunk0, play_emb, hand_emb, W1, b1, W2, b2):
    Bc = S.shape[0]
    BT = 512
    co = chunk0 // BT
    return pl.pallas_call(
        _mlp_body,
        grid=(Bc // BT,),
        in_specs=[
            pl.BlockSpec((BT, _COLS), lambda i: (i, 0)),
            pl.BlockSpec((BT, 222), lambda i: (i + co, 0)),
            pl.BlockSpec((56, 12), lambda i: (0, 0)),
            pl.BlockSpec((54, 20), lambda i: (0, 0)),
            pl.BlockSpec((128, 56), lambda i: (0, 0)),
            pl.BlockSpec((1, 128), lambda i: (0, 0)),
            pl.BlockSpec((55, 128), lambda i: (0, 0)),
            pl.BlockSpec((1, 55), lambda i: (0, 0)),
        ],
        out_specs=pl.BlockSpec((BT, 55), lambda i: (i, 0)),
        out_shape=jax.ShapeDtypeStruct((Bc, 55), jnp.float32),
    )(S, x, play_emb, hand_emb, W1, b1.reshape(1, 128), W2,
      b2.reshape(1, 55))


def kernel(x, play_emb, hand_emb, W1, b1, W2, b2):
    B = x.shape[0]
    n_chunk = 2
    cb = B // n_chunk
    outs = []
    for i in range(n_chunk):
        S = _sc_histogram(x, i * cb, cb)
        outs.append(_tc_mlp(S, x, i * cb, play_emb, hand_emb, W1, b1, W2, b2))
    return jnp.concatenate(outs, axis=0)


# speedup vs baseline: 1.5833x; 1.0195x over previous
import functools

import jax
import jax.numpy as jnp
from jax import lax
from jax.experimental import pallas as pl
from jax.experimental.pallas import tpu as pltpu
from jax.experimental.pallas import tpu_sc as plsc

_COLS = 224
_SUB = 128


def _sc_histogram(x, base, nrows):
    _, C = x.shape
    info = plsc.get_sparse_core_info()
    nw = info.num_cores * info.num_subcores
    per_w = nrows // nw
    n_sub = per_w // _SUB
    mesh = plsc.VectorSubcoreMesh(core_axis_name="c", subcore_axis_name="s")

    @functools.partial(
        pl.kernel,
        out_type=jax.ShapeDtypeStruct((nrows, _COLS), jnp.float32),
        mesh=mesh,
        scratch_types=[
            pltpu.VMEM((64, C), jnp.int32),
            pltpu.VMEM((64, C), jnp.int32),
            pltpu.VMEM((_SUB, _COLS), jnp.float32),
            pltpu.VMEM((_SUB, _COLS), jnp.float32),
            pltpu.SemaphoreType.DMA,
            pltpu.SemaphoreType.DMA,
            pltpu.SemaphoreType.DMA,
            pltpu.SemaphoreType.DMA,
        ],
        compiler_params=pltpu.CompilerParams(use_tc_tiling_on_sc=True,
                                             needs_layout_passes=False),
    )
    def k(x_hbm, out_hbm, xs_a, xs_b, cnt_a, cnt_b,
          isem_a, isem_b, osem_a, osem_b):
        wid = lax.axis_index("s") * info.num_cores + lax.axis_index("c")
        lanes = lax.iota(jnp.int32, 16)
        ones = jnp.full((16,), 1.0, jnp.float32)
        zeros16 = jnp.zeros((16,), jnp.float32)
        xbufs, isems = (xs_a, xs_b), (isem_a, isem_b)
        cbufs, osems = (cnt_a, cnt_b), (osem_a, osem_b)
        pend_in = [None, None]
        pend_out = [None, None]
        n_half = per_w // 64

        def x_copy(hidx, buf, sem):
            row0 = pl.multiple_of(base + wid * per_w + hidx * 64, 64)
            return pltpu.async_copy(x_hbm.at[pl.ds(row0, 64), :], buf, sem)

        pend_in[0] = x_copy(0, xbufs[0], isems[0])

        for sc_i in range(n_sub):
            cnt, osem = cbufs[sc_i % 2], osems[sc_i % 2]
            row0 = pl.multiple_of(wid * per_w + sc_i * _SUB, _SUB)
            if pend_out[sc_i % 2] is not None:
                pend_out[sc_i % 2].wait()

            @plsc.parallel_loop(0, _SUB, unroll=4)
            def _(r, cnt=cnt):
                for cg in range(224 // 16):
                    cnt[r, pl.ds(cg * 16, 16)] = zeros16

            for half in range(2):
                hidx = sc_i * 2 + half
                xs = xbufs[hidx % 2]
                pend_in[hidx % 2].wait()
                if hidx + 1 < n_half:
                    nb = (hidx + 1) % 2
                    pend_in[nb] = x_copy(hidx + 1, xbufs[nb], isems[nb])
                groups = [(g * 16 + lanes, half * 64 + g * 16 + lanes)
                          for g in range(4)]
                for seg in range(4):
                    boff = 56 * seg
                    hi = min(boff + 56, C)

                    @plsc.parallel_loop(boff, hi, unroll=(4 if seg < 3 else 2))
                    def _(j, xs=xs, cnt=cnt, boff=boff, groups=groups):
                        jv = jnp.full((16,), j, jnp.int32)
                        idxs = [plsc.load_gather(xs, [rows, jv])
                                for rows, _ in groups]
                        for (_, crows), idx in zip(groups, idxs):
                            plsc.addupdate_scatter(cnt, [crows, idx + boff],
                                                   ones)

            pend_out[sc_i % 2] = pltpu.async_copy(
                cnt, out_hbm.at[pl.ds(row0, _SUB), :], osem)
        for p in pend_out:
            if p is not None:
                p.wait()

    return k(x)


def _mlp_body(s_ref, x_ref, pe_ref, he_ref, w1_ref, b1_ref, w2_ref, b2_ref,
              o_ref):
    bt = s_ref.shape[0]
    pe = pe_ref[...]
    he = he_ref[...]
    w1 = w1_ref[...]
    f32 = jnp.float32
    tt = (((1,), (1,)), ((), ()))
    nn = (((1,), (0,)), ((), ()))
    A1 = lax.dot_general(pe, w1[:, 0:12], tt, preferred_element_type=f32)
    A2 = lax.dot_general(pe, w1[:, 12:24], tt, preferred_element_type=f32)
    A3 = lax.dot_general(pe, w1[:, 24:36], tt, preferred_element_type=f32)
    A4 = lax.dot_general(he, w1[:, 36:56], tt, preferred_element_type=f32)
    s = s_ref[...]
    hp = (lax.dot_general(s[:, 0:56], A1, nn, preferred_element_type=f32)
          + lax.dot_general(s[:, 56:112], A2, nn, preferred_element_type=f32)
          + lax.dot_general(s[:, 112:168], A3, nn, preferred_element_type=f32)
          + lax.dot_general(s[:, 168:222], A4[0:54], nn,
                            preferred_element_type=f32)
          + b1_ref[...])
    h = jnp.maximum(hp, 0.0)
    out = lax.dot_general(h, w2_ref[...], tt, preferred_element_type=f32)
    a4 = x_ref[:, 168:222].astype(f32)
    skip = jnp.concatenate([a4, jnp.zeros((bt, 1), jnp.float32)], axis=1)
    o_ref[...] = out + skip + b2_ref[...]


def _tc_mlp(S, x, chunk0, play_emb, hand_emb, W1, b1, W2, b2):
    Bc = S.shape[0]
    BT = 512
    co = chunk0 // BT
    return pl.pallas_call(
        _mlp_body,
        grid=(Bc // BT,),
        in_specs=[
            pl.BlockSpec((BT, _COLS), lambda i: (i, 0)),
            pl.BlockSpec((BT, 222), lambda i: (i + co, 0)),
            pl.BlockSpec((56, 12), lambda i: (0, 0)),
            pl.BlockSpec((54, 20), lambda i: (0, 0)),
            pl.BlockSpec((128, 56), lambda i: (0, 0)),
            pl.BlockSpec((1, 128), lambda i: (0, 0)),
            pl.BlockSpec((55, 128), lambda i: (0, 0)),
            pl.BlockSpec((1, 55), lambda i: (0, 0)),
        ],
        out_specs=pl.BlockSpec((BT, 55), lambda i: (i, 0)),
        out_shape=jax.ShapeDtypeStruct((Bc, 55), jnp.float32),
    )(S, x, play_emb, hand_emb, W1, b1.reshape(1, 128), W2,
      b2.reshape(1, 55))


def kernel(x, play_emb, hand_emb, W1, b1, W2, b2):
    B = x.shape[0]
    chunks = [B // 2, B // 4, B // 4] if B % 4 == 0 else [B]
    outs = []
    base = 0
    for cb in chunks:
        S = _sc_histogram(x, base, cb)
        outs.append(_tc_mlp(S, x, base, play_emb, hand_emb, W1, b1, W2, b2))
        base += cb
    return jnp.concatenate(outs, axis=0)
